# baseline (speedup 1.0000x reference)
import functools

import jax
import jax.numpy as jnp
from jax import lax
from jax.experimental import pallas as pl
from jax.experimental.pallas import tpu as pltpu
from jax.experimental.pallas import tpu_sc as plsc

N = 10000
E = 320000
DIN = 128
DE = 16
H = 128
L = 3
NOUT = 2

NP = 10240
CHUNK = 128
NWORK = 32
CPW = 80
EP = NWORK * CPW * CHUNK
SENT = N


def _leaky(v):
    return jnp.where(v > 0, v, 0.01 * v)


def _prep_body(deg_p_ref, x_ref, encw_ref, encb_ref, w0_ref, g0_ref, dinv_ref):
    deg = deg_p_ref[0, :, 0:1] + deg_p_ref[1, :, 0:1] + 1.0
    dinv = lax.rsqrt(deg)
    dinv_ref[...] = dinv
    h0 = jnp.dot(x_ref[...], encw_ref[...],
                 preferred_element_type=jnp.float32) + encb_ref[...]
    g = jnp.dot(h0, w0_ref[...], preferred_element_type=jnp.float32)
    g0_ref[0:N, :] = g * dinv[0:N]
    g0_ref[N:NP, :] = jnp.zeros((NP - N, H), jnp.float32)


def _tc_prep(deg_p, x, enc_W, enc_b, W0):
    return pl.pallas_call(
        _prep_body,
        out_shape=(
            jax.ShapeDtypeStruct((NP, H), jnp.float32),
            jax.ShapeDtypeStruct((NP, 1), jnp.float32),
        ),
    )(deg_p, x, enc_W, enc_b.reshape(1, H), W0)


def _combine_body(p_ref, g_ref, dinv_ref, cb_ref, lng_ref, lnb_ref, wn_ref,
                  gn_ref):
    dinv = dinv_ref[...]
    v = dinv * (p_ref[0] + p_ref[1] + g_ref[...]) + cb_ref[...]
    m = v.mean(-1, keepdims=True)
    var = ((v - m) ** 2).mean(-1, keepdims=True)
    h = _leaky((v - m) / jnp.sqrt(var + 1e-5) * lng_ref[...] + lnb_ref[...])
    g = jnp.dot(h, wn_ref[...], preferred_element_type=jnp.float32) * dinv
    gn_ref[0:N, :] = g[0:N]
    gn_ref[N:NP, :] = jnp.zeros((NP - N, H), jnp.float32)


def _tc_combine(p, g, dinv, conv_b, ln_g, ln_b, W_next):
    return pl.pallas_call(
        _combine_body,
        out_shape=jax.ShapeDtypeStruct((NP, H), jnp.float32),
    )(p, g, dinv, conv_b.reshape(1, H), ln_g.reshape(1, H),
      ln_b.reshape(1, H), W_next)


def _final_body(p_ref, g_ref, dinv_ref, cb_ref, lng_ref, lnb_ref,
                wa_ref, wb_ref, no1w_ref, no1b_ref, no2w_ref, no2b_ref,
                a_ref, b_ref, nout_ref):
    dinv = dinv_ref[...]
    v = dinv * (p_ref[0] + p_ref[1] + g_ref[...]) + cb_ref[...]
    m = v.mean(-1, keepdims=True)
    var = ((v - m) ** 2).mean(-1, keepdims=True)
    h = _leaky((v - m) / jnp.sqrt(var + 1e-5) * lng_ref[...] + lnb_ref[...])
    a_ref[...] = jnp.dot(h, wa_ref[...], preferred_element_type=jnp.float32)
    b_ref[...] = jnp.dot(h, wb_ref[...], preferred_element_type=jnp.float32)
    z = _leaky(jnp.dot(h[0:N], no1w_ref[...],
                       preferred_element_type=jnp.float32) + no1b_ref[...])
    nout_ref[...] = jnp.tanh(
        jnp.dot(z, no2w_ref[...], preferred_element_type=jnp.float32)
        + no2b_ref[...])


def _tc_final(p, g, dinv, conv_b, ln_g, ln_b, Wa, Wb, no1_W, no1_b, no2_W,
              no2_b):
    return pl.pallas_call(
        _final_body,
        out_shape=(
            jax.ShapeDtypeStruct((NP, H), jnp.float32),
            jax.ShapeDtypeStruct((NP, H), jnp.float32),
            jax.ShapeDtypeStruct((N, NOUT), jnp.float32),
        ),
    )(p, g, dinv, conv_b.reshape(1, H), ln_g.reshape(1, H), ln_b.reshape(1, H),
      Wa, Wb, no1_W, no1_b.reshape(1, H), no2_W, no2_b.reshape(1, NOUT))


_EBLK = CPW * CHUNK


def _edge_body(sa_ref, sb_ref, attr_ref, we_ref, b1_ref, w2_ref, b2_ref,
               out_ref):
    c = jnp.dot(attr_ref[...], we_ref[...], preferred_element_type=jnp.float32)
    z = _leaky(sa_ref[...] + sb_ref[...] + c + b1_ref[...])
    w = jnp.dot(z, w2_ref[...], preferred_element_type=jnp.float32)
    out_ref[...] = jnp.tanh(w + b2_ref[...])


def _tc_edge(SA, SB, attr_p, We, eo1_b, eo2_W, eo2_b):
    return pl.pallas_call(
        _edge_body,
        grid=(EP // _EBLK,),
        in_specs=[
            pl.BlockSpec((_EBLK, H), lambda i: (i, 0)),
            pl.BlockSpec((_EBLK, H), lambda i: (i, 0)),
            pl.BlockSpec((_EBLK, DE), lambda i: (i, 0)),
            pl.BlockSpec((DE, H), lambda i: (0, 0)),
            pl.BlockSpec((1, H), lambda i: (0, 0)),
            pl.BlockSpec((H, 1), lambda i: (0, 0)),
            pl.BlockSpec((1, 1), lambda i: (0, 0)),
        ],
        out_specs=pl.BlockSpec((_EBLK, 1), lambda i: (i, 0)),
        out_shape=jax.ShapeDtypeStruct((EP, 1), jnp.float32),
    )(SA, SB, attr_p, We, eo1_b.reshape(1, H), eo2_W, eo2_b.reshape(1, 1))


_VMESH = plsc.VectorSubcoreMesh(core_axis_name="c", subcore_axis_name="s")
NPT = NP // 16
NROW = NWORK * CPW

CPW0 = 40
CPW1 = 2 * CPW - CPW0
CPWMAX = max(CPW0, CPW1)


def _worker_split(c, s):
    cpw = jnp.where(c == 0, CPW0, CPW1)
    base = jnp.where(c == 0, s * CPW0, 16 * CPW0 + s * CPW1)
    return cpw, pl.multiple_of(base, 8)


def _deg_body(dst_hbm, ones_hbm, zero_hbm, out_hbm, dstv, ones, shared,
              ss0, ss1):
    c = lax.axis_index("c")
    s = lax.axis_index("s")
    wid = s * 2 + c
    pltpu.sync_copy(dst_hbm.at[pl.ds(wid * CPW, CPW)], dstv)
    pltpu.sync_copy(ones_hbm, ones)
    pltpu.sync_copy(zero_hbm, shared.at[pl.ds(s * NPT, NPT)])

    plsc.subcore_barrier()

    @pl.loop(0, CPW, step=2)
    def _(j):
        s0 = pltpu.async_copy(ones, shared.at[dstv.at[j]], ss0, add=True)
        s1 = pltpu.async_copy(ones, shared.at[dstv.at[j + 1]], ss1, add=True)
        s0.wait()
        s1.wait()

    plsc.subcore_barrier()
    pltpu.sync_copy(shared.at[pl.ds(s * NPT, NPT)],
                    out_hbm.at[c, pl.ds(s * NPT, NPT)])


def _sc_degree(dst_m):
    f = functools.partial(
        pl.kernel,
        out_type=jax.ShapeDtypeStruct((2, NP, H), jnp.float32),
        mesh=_VMESH,
        scratch_types=[
            pltpu.VMEM((CPW, CHUNK), jnp.int32),
            pltpu.VMEM((CHUNK, H), jnp.float32),
            pltpu.VMEM_SHARED((NP, H), jnp.float32),
            pltpu.SemaphoreType.DMA,
            pltpu.SemaphoreType.DMA,
        ],
    )(_deg_body)
    return f(dst_m, jnp.ones((CHUNK, H), jnp.float32),
             jnp.zeros((NPT, H), jnp.float32))


def _agg_body(g_hbm, src_hbm, dst_hbm, zero_hbm, out_hbm, is0, is1, dstv,
              r0, r1, shared, gs0, gs1, ss0, ss1):
    c = lax.axis_index("c")
    s = lax.axis_index("s")
    cpw, base = _worker_split(c, s)
    pltpu.sync_copy(dst_hbm.at[pl.ds(base, CPWMAX)], dstv)
    pltpu.sync_copy(zero_hbm, shared.at[pl.ds(s * NPT, NPT)])

    plsc.subcore_barrier()

    pltpu.sync_copy(src_hbm.at[pl.ds(base * CHUNK, CHUNK)], is0)
    pltpu.async_copy(g_hbm.at[is0], r0, gs0)
    pltpu.sync_copy(src_hbm.at[pl.ds((base + 1) * CHUNK, CHUNK)], is1)
    pltpu.async_copy(g_hbm.at[is1], r1, gs1)

    @pl.loop(0, cpw, step=2)
    def _(j):
        pltpu.make_async_copy(g_hbm.at[is0], r0, gs0).wait()
        s0 = pltpu.async_copy(r0, shared.at[dstv.at[j]], ss0, add=True)
        pltpu.make_async_copy(g_hbm.at[is1], r1, gs1).wait()
        s1 = pltpu.async_copy(r1, shared.at[dstv.at[j + 1]], ss1, add=True)

        @pl.when(j + 2 < cpw)
        def _():
            pltpu.sync_copy(src_hbm.at[pl.ds((base + j + 2) * CHUNK, CHUNK)],
                            is0)
            s0.wait()
            pltpu.async_copy(g_hbm.at[is0], r0, gs0)
            pltpu.sync_copy(src_hbm.at[pl.ds((base + j + 3) * CHUNK, CHUNK)],
                            is1)
            s1.wait()
            pltpu.async_copy(g_hbm.at[is1], r1, gs1)

    pltpu.make_async_copy(r0, shared.at[dstv.at[cpw - 2]], ss0).wait()
    pltpu.make_async_copy(r1, shared.at[dstv.at[cpw - 1]], ss1).wait()

    plsc.subcore_barrier()
    pltpu.sync_copy(shared.at[pl.ds(s * NPT, NPT)],
                    out_hbm.at[c, pl.ds(s * NPT, NPT)])


def _sc_layer_agg(g, src_f, dst_m):
    f = functools.partial(
        pl.kernel,
        out_type=jax.ShapeDtypeStruct((2, NP, H), jnp.float32),
        mesh=_VMESH,
        scratch_types=[
            pltpu.VMEM((CHUNK,), jnp.int32),
            pltpu.VMEM((CHUNK,), jnp.int32),
            pltpu.VMEM((CPWMAX, CHUNK), jnp.int32),
            pltpu.VMEM((CHUNK, H), jnp.float32),
            pltpu.VMEM((CHUNK, H), jnp.float32),
            pltpu.VMEM_SHARED((NP, H), jnp.float32),
            pltpu.SemaphoreType.DMA,
            pltpu.SemaphoreType.DMA,
            pltpu.SemaphoreType.DMA,
            pltpu.SemaphoreType.DMA,
        ],
    )(_agg_body)
    return f(g, src_f, dst_m, jnp.zeros((NPT, H), jnp.float32))


def _egather_body(a_hbm, b_hbm, src_hbm, dst_hbm, sa_hbm, sb_hbm,
                  srcv, dstv, ra0, rb0, ra1, rb1,
                  ga0, gb0, ga1, gb1, wa0, wb0, wa1, wb1):
    c = lax.axis_index("c")
    s = lax.axis_index("s")
    cpw, base = _worker_split(c, s)
    pltpu.sync_copy(src_hbm.at[pl.ds(base, CPWMAX)], srcv)
    pltpu.sync_copy(dst_hbm.at[pl.ds(base, CPWMAX)], dstv)

    pltpu.async_copy(a_hbm.at[srcv.at[0]], ra0, ga0)
    pltpu.async_copy(b_hbm.at[dstv.at[0]], rb0, gb0)
    pltpu.async_copy(a_hbm.at[srcv.at[1]], ra1, ga1)
    pltpu.async_copy(b_hbm.at[dstv.at[1]], rb1, gb1)

    @pl.loop(0, cpw, step=2)
    def _(j):
        o0 = pl.ds((base + j) * CHUNK, CHUNK)
        o1 = pl.ds((base + j + 1) * CHUNK, CHUNK)
        pltpu.make_async_copy(a_hbm.at[srcv.at[j]], ra0, ga0).wait()
        w0 = pltpu.async_copy(ra0, sa_hbm.at[o0], wa0)
        pltpu.make_async_copy(b_hbm.at[dstv.at[j]], rb0, gb0).wait()
        w1 = pltpu.async_copy(rb0, sb_hbm.at[o0], wb0)
        pltpu.make_async_copy(a_hbm.at[srcv.at[j + 1]], ra1, ga1).wait()
        w2 = pltpu.async_copy(ra1, sa_hbm.at[o1], wa1)
        pltpu.make_async_copy(b_hbm.at[dstv.at[j + 1]], rb1, gb1).wait()
        w3 = pltpu.async_copy(rb1, sb_hbm.at[o1], wb1)

        @pl.when(j + 2 < cpw)
        def _():
            w0.wait()
            pltpu.async_copy(a_hbm.at[srcv.at[j + 2]], ra0, ga0)
            w1.wait()
            pltpu.async_copy(b_hbm.at[dstv.at[j + 2]], rb0, gb0)
            w2.wait()
            pltpu.async_copy(a_hbm.at[srcv.at[j + 3]], ra1, ga1)
            w3.wait()
            pltpu.async_copy(b_hbm.at[dstv.at[j + 3]], rb1, gb1)

    oz0 = pl.ds((base + cpw - 2) * CHUNK, CHUNK)
    oz1 = pl.ds((base + cpw - 1) * CHUNK, CHUNK)
    pltpu.make_async_copy(ra0, sa_hbm.at[oz0], wa0).wait()
    pltpu.make_async_copy(rb0, sb_hbm.at[oz0], wb0).wait()
    pltpu.make_async_copy(ra1, sa_hbm.at[oz1], wa1).wait()
    pltpu.make_async_copy(rb1, sb_hbm.at[oz1], wb1).wait()


def _sc_edge_gather(A, B, src_m, dst_m):
    f = functools.partial(
        pl.kernel,
        out_type=(jax.ShapeDtypeStruct((EP, H), jnp.float32),
                  jax.ShapeDtypeStruct((EP, H), jnp.float32)),
        mesh=_VMESH,
        scratch_types=[
            pltpu.VMEM((CPWMAX, CHUNK), jnp.int32),
            pltpu.VMEM((CPWMAX, CHUNK), jnp.int32),
            pltpu.VMEM((CHUNK, H), jnp.float32),
            pltpu.VMEM((CHUNK, H), jnp.float32),
            pltpu.VMEM((CHUNK, H), jnp.float32),
            pltpu.VMEM((CHUNK, H), jnp.float32),
        ] + [pltpu.SemaphoreType.DMA] * 8,
    )(_egather_body)
    return f(A, B, src_m, dst_m)


def kernel(x, edge_index, edge_attr, enc_W, enc_b, conv_W, conv_b, ln_g, ln_b,
           no1_W, no1_b, no2_W, no2_b, eo1_W, eo1_b, eo2_W, eo2_b):
    src = edge_index[0]
    dst = edge_index[1]
    pad = jnp.full((EP - E,), SENT, jnp.int32)
    src_f = jnp.concatenate([src, pad])
    src_m = src_f.reshape(NROW, CHUNK)
    dst_m = jnp.concatenate([dst, pad]).reshape(NROW, CHUNK)
    attr_p = jnp.concatenate(
        [edge_attr, jnp.zeros((EP - E, DE), jnp.float32)], axis=0)

    deg_p = _sc_degree(dst_m)
    g, dinv = _tc_prep(deg_p, x, enc_W, enc_b, conv_W[0])
    for i in range(L - 1):
        p = _sc_layer_agg(g, src_f, dst_m)
        g = _tc_combine(p, g, dinv, conv_b[i], ln_g[i], ln_b[i], conv_W[i + 1])
    p = _sc_layer_agg(g, src_f, dst_m)
    A, B, n_out = _tc_final(p, g, dinv, conv_b[L - 1], ln_g[L - 1],
                            ln_b[L - 1], eo1_W[0:H], eo1_W[H:2 * H],
                            no1_W, no1_b, no2_W, no2_b)
    SA, SB = _sc_edge_gather(A, B, src_m, dst_m)
    e_out = _tc_edge(SA, SB, attr_p, eo1_W[2 * H:], eo1_b, eo2_W, eo2_b)
    return (e_out[0:E, 0], n_out)


# speedup vs baseline: 1.0637x; 1.0637x over previous
import functools

import jax
import jax.numpy as jnp
from jax import lax
from jax.experimental import pallas as pl
from jax.experimental.pallas import tpu as pltpu
from jax.experimental.pallas import tpu_sc as plsc

N = 10000
E = 320000
DIN = 128
DE = 16
H = 128
L = 3
NOUT = 2

NP = 10240
CHUNK = 128
NWORK = 32
CPW = 80
EP = NWORK * CPW * CHUNK
SENT = N


def _leaky(v):
    return jnp.where(v > 0, v, 0.01 * v)


def _prep_body(deg_p_ref, x_ref, encw_ref, encb_ref, w0_ref, g0_ref, dinv_ref):
    deg = deg_p_ref[0, :, 0:1] + deg_p_ref[1, :, 0:1] + 1.0
    dinv = lax.rsqrt(deg)
    dinv_ref[...] = dinv
    h0 = jnp.dot(x_ref[...], encw_ref[...],
                 preferred_element_type=jnp.float32) + encb_ref[...]
    g = jnp.dot(h0, w0_ref[...], preferred_element_type=jnp.float32)
    g0_ref[0:N, :] = g * dinv[0:N]
    g0_ref[N:NP, :] = jnp.zeros((NP - N, H), jnp.float32)


def _tc_prep(deg_p, x, enc_W, enc_b, W0):
    return pl.pallas_call(
        _prep_body,
        out_shape=(
            jax.ShapeDtypeStruct((NP, H), jnp.float32),
            jax.ShapeDtypeStruct((NP, 1), jnp.float32),
        ),
    )(deg_p, x, enc_W, enc_b.reshape(1, H), W0)


def _combine_body(p_ref, g_ref, dinv_ref, cb_ref, lng_ref, lnb_ref, wn_ref,
                  gn_ref):
    dinv = dinv_ref[...]
    v = dinv * (p_ref[0] + p_ref[1] + g_ref[...]) + cb_ref[...]
    m = v.mean(-1, keepdims=True)
    var = ((v - m) ** 2).mean(-1, keepdims=True)
    h = _leaky((v - m) / jnp.sqrt(var + 1e-5) * lng_ref[...] + lnb_ref[...])
    g = jnp.dot(h, wn_ref[...], preferred_element_type=jnp.float32) * dinv
    gn_ref[0:N, :] = g[0:N]
    gn_ref[N:NP, :] = jnp.zeros((NP - N, H), jnp.float32)


def _tc_combine(p, g, dinv, conv_b, ln_g, ln_b, W_next):
    return pl.pallas_call(
        _combine_body,
        out_shape=jax.ShapeDtypeStruct((NP, H), jnp.float32),
    )(p, g, dinv, conv_b.reshape(1, H), ln_g.reshape(1, H),
      ln_b.reshape(1, H), W_next)


def _final_body(p_ref, g_ref, dinv_ref, cb_ref, lng_ref, lnb_ref,
                wa_ref, wb_ref, no1w_ref, no1b_ref, no2w_ref, no2b_ref,
                a_ref, b_ref, nout_ref):
    dinv = dinv_ref[...]
    v = dinv * (p_ref[0] + p_ref[1] + g_ref[...]) + cb_ref[...]
    m = v.mean(-1, keepdims=True)
    var = ((v - m) ** 2).mean(-1, keepdims=True)
    h = _leaky((v - m) / jnp.sqrt(var + 1e-5) * lng_ref[...] + lnb_ref[...])
    a_ref[...] = jnp.dot(h, wa_ref[...], preferred_element_type=jnp.float32)
    b_ref[...] = jnp.dot(h, wb_ref[...], preferred_element_type=jnp.float32)
    z = _leaky(jnp.dot(h[0:N], no1w_ref[...],
                       preferred_element_type=jnp.float32) + no1b_ref[...])
    nout_ref[...] = jnp.tanh(
        jnp.dot(z, no2w_ref[...], preferred_element_type=jnp.float32)
        + no2b_ref[...])


def _tc_final(p, g, dinv, conv_b, ln_g, ln_b, Wa, Wb, no1_W, no1_b, no2_W,
              no2_b):
    return pl.pallas_call(
        _final_body,
        out_shape=(
            jax.ShapeDtypeStruct((NP, H), jnp.float32),
            jax.ShapeDtypeStruct((NP, H), jnp.float32),
            jax.ShapeDtypeStruct((N, NOUT), jnp.float32),
        ),
    )(p, g, dinv, conv_b.reshape(1, H), ln_g.reshape(1, H), ln_b.reshape(1, H),
      Wa, Wb, no1_W, no1_b.reshape(1, H), no2_W, no2_b.reshape(1, NOUT))


_EBLK = CPW * CHUNK


def _edge_body(sa_ref, sb_ref, attr_ref, we_ref, b1_ref, w2_ref, b2_ref,
               out_ref):
    c = jnp.dot(attr_ref[...], we_ref[...], preferred_element_type=jnp.float32)
    sab = sa_ref[...].astype(jnp.float32) + sb_ref[...].astype(jnp.float32)
    z = _leaky(sab + c + b1_ref[...])
    w = jnp.dot(z, w2_ref[...], preferred_element_type=jnp.float32)
    out_ref[...] = jnp.tanh(w + b2_ref[...])


def _tc_edge(SA, SB, attr_p, We, eo1_b, eo2_W, eo2_b):
    return pl.pallas_call(
        _edge_body,
        grid=(EP // _EBLK,),
        in_specs=[
            pl.BlockSpec((_EBLK, H), lambda i: (i, 0)),
            pl.BlockSpec((_EBLK, H), lambda i: (i, 0)),
            pl.BlockSpec((_EBLK, DE), lambda i: (i, 0)),
            pl.BlockSpec((DE, H), lambda i: (0, 0)),
            pl.BlockSpec((1, H), lambda i: (0, 0)),
            pl.BlockSpec((H, 1), lambda i: (0, 0)),
            pl.BlockSpec((1, 1), lambda i: (0, 0)),
        ],
        out_specs=pl.BlockSpec((_EBLK, 1), lambda i: (i, 0)),
        out_shape=jax.ShapeDtypeStruct((EP, 1), jnp.float32),
    )(SA, SB, attr_p, We, eo1_b.reshape(1, H), eo2_W, eo2_b.reshape(1, 1))


_VMESH = plsc.VectorSubcoreMesh(core_axis_name="c", subcore_axis_name="s")
NPT = NP // 16
NROW = NWORK * CPW

CPW0 = 80
CPW1 = 2 * CPW - CPW0
CPWMAX = max(CPW0, CPW1)


def _worker_split(c, s):
    cpw = jnp.where(c == 0, CPW0, CPW1)
    base = jnp.where(c == 0, s * CPW0, 16 * CPW0 + s * CPW1)
    return cpw, pl.multiple_of(base, 8)


def _deg_body(dst_hbm, ones_hbm, zero_hbm, out_hbm, dstv, ones, shared,
              ss0, ss1):
    c = lax.axis_index("c")
    s = lax.axis_index("s")
    wid = s * 2 + c
    pltpu.sync_copy(dst_hbm.at[pl.ds(wid * CPW, CPW)], dstv)
    pltpu.sync_copy(ones_hbm, ones)
    pltpu.sync_copy(zero_hbm, shared.at[pl.ds(s * NPT, NPT)])

    plsc.subcore_barrier()

    @pl.loop(0, CPW, step=2)
    def _(j):
        s0 = pltpu.async_copy(ones, shared.at[dstv.at[j]], ss0, add=True)
        s1 = pltpu.async_copy(ones, shared.at[dstv.at[j + 1]], ss1, add=True)
        s0.wait()
        s1.wait()

    plsc.subcore_barrier()
    pltpu.sync_copy(shared.at[pl.ds(s * NPT, NPT)],
                    out_hbm.at[c, pl.ds(s * NPT, NPT)])


def _sc_degree(dst_m):
    f = functools.partial(
        pl.kernel,
        out_type=jax.ShapeDtypeStruct((2, NP, H), jnp.float32),
        mesh=_VMESH,
        scratch_types=[
            pltpu.VMEM((CPW, CHUNK), jnp.int32),
            pltpu.VMEM((CHUNK, H), jnp.float32),
            pltpu.VMEM_SHARED((NP, H), jnp.float32),
            pltpu.SemaphoreType.DMA,
            pltpu.SemaphoreType.DMA,
        ],
    )(_deg_body)
    return f(dst_m, jnp.ones((CHUNK, H), jnp.float32),
             jnp.zeros((NPT, H), jnp.float32))


def _agg_body(g_hbm, src_hbm, dst_hbm, zero_hbm, out_hbm, is0, is1, dstv,
              r0, r1, shared, gs0, gs1, ss0, ss1):
    c = lax.axis_index("c")
    s = lax.axis_index("s")
    cpw, base = _worker_split(c, s)
    pltpu.sync_copy(dst_hbm.at[pl.ds(base, CPWMAX)], dstv)
    pltpu.sync_copy(zero_hbm, shared.at[pl.ds(s * NPT, NPT)])

    plsc.subcore_barrier()

    pltpu.sync_copy(src_hbm.at[pl.ds(base * CHUNK, CHUNK)], is0)
    pltpu.async_copy(g_hbm.at[is0], r0, gs0)
    pltpu.sync_copy(src_hbm.at[pl.ds((base + 1) * CHUNK, CHUNK)], is1)
    pltpu.async_copy(g_hbm.at[is1], r1, gs1)

    @pl.loop(0, cpw, step=2)
    def _(j):
        pltpu.make_async_copy(g_hbm.at[is0], r0, gs0).wait()
        s0 = pltpu.async_copy(r0, shared.at[dstv.at[j]], ss0, add=True)
        pltpu.make_async_copy(g_hbm.at[is1], r1, gs1).wait()
        s1 = pltpu.async_copy(r1, shared.at[dstv.at[j + 1]], ss1, add=True)

        @pl.when(j + 2 < cpw)
        def _():
            pltpu.sync_copy(src_hbm.at[pl.ds((base + j + 2) * CHUNK, CHUNK)],
                            is0)
            s0.wait()
            pltpu.async_copy(g_hbm.at[is0], r0, gs0)
            pltpu.sync_copy(src_hbm.at[pl.ds((base + j + 3) * CHUNK, CHUNK)],
                            is1)
            s1.wait()
            pltpu.async_copy(g_hbm.at[is1], r1, gs1)

    pltpu.make_async_copy(r0, shared.at[dstv.at[cpw - 2]], ss0).wait()
    pltpu.make_async_copy(r1, shared.at[dstv.at[cpw - 1]], ss1).wait()

    plsc.subcore_barrier()
    pltpu.sync_copy(shared.at[pl.ds(s * NPT, NPT)],
                    out_hbm.at[c, pl.ds(s * NPT, NPT)])


def _sc_layer_agg(g, src_f, dst_m):
    f = functools.partial(
        pl.kernel,
        out_type=jax.ShapeDtypeStruct((2, NP, H), jnp.float32),
        mesh=_VMESH,
        scratch_types=[
            pltpu.VMEM((CHUNK,), jnp.int32),
            pltpu.VMEM((CHUNK,), jnp.int32),
            pltpu.VMEM((CPWMAX, CHUNK), jnp.int32),
            pltpu.VMEM((CHUNK, H), jnp.float32),
            pltpu.VMEM((CHUNK, H), jnp.float32),
            pltpu.VMEM_SHARED((NP, H), jnp.float32),
            pltpu.SemaphoreType.DMA,
            pltpu.SemaphoreType.DMA,
            pltpu.SemaphoreType.DMA,
            pltpu.SemaphoreType.DMA,
        ],
    )(_agg_body)
    return f(g, src_f, dst_m, jnp.zeros((NPT, H), jnp.float32))


def _egather_body(a_hbm, b_hbm, src_hbm, dst_hbm, sa_hbm, sb_hbm,
                  srcv, dstv, ra0, rb0, ra1, rb1,
                  ga0, gb0, ga1, gb1, wa0, wb0, wa1, wb1):
    c = lax.axis_index("c")
    s = lax.axis_index("s")
    cpw, base = _worker_split(c, s)
    pltpu.sync_copy(src_hbm.at[pl.ds(base, CPWMAX)], srcv)
    pltpu.sync_copy(dst_hbm.at[pl.ds(base, CPWMAX)], dstv)

    pltpu.async_copy(a_hbm.at[srcv.at[0]], ra0, ga0)
    pltpu.async_copy(b_hbm.at[dstv.at[0]], rb0, gb0)
    pltpu.async_copy(a_hbm.at[srcv.at[1]], ra1, ga1)
    pltpu.async_copy(b_hbm.at[dstv.at[1]], rb1, gb1)

    @pl.loop(0, cpw, step=2)
    def _(j):
        o0 = pl.ds((base + j) * CHUNK, CHUNK)
        o1 = pl.ds((base + j + 1) * CHUNK, CHUNK)
        pltpu.make_async_copy(a_hbm.at[srcv.at[j]], ra0, ga0).wait()
        w0 = pltpu.async_copy(ra0, sa_hbm.at[o0], wa0)
        pltpu.make_async_copy(b_hbm.at[dstv.at[j]], rb0, gb0).wait()
        w1 = pltpu.async_copy(rb0, sb_hbm.at[o0], wb0)
        pltpu.make_async_copy(a_hbm.at[srcv.at[j + 1]], ra1, ga1).wait()
        w2 = pltpu.async_copy(ra1, sa_hbm.at[o1], wa1)
        pltpu.make_async_copy(b_hbm.at[dstv.at[j + 1]], rb1, gb1).wait()
        w3 = pltpu.async_copy(rb1, sb_hbm.at[o1], wb1)

        @pl.when(j + 2 < cpw)
        def _():
            w0.wait()
            pltpu.async_copy(a_hbm.at[srcv.at[j + 2]], ra0, ga0)
            w1.wait()
            pltpu.async_copy(b_hbm.at[dstv.at[j + 2]], rb0, gb0)
            w2.wait()
            pltpu.async_copy(a_hbm.at[srcv.at[j + 3]], ra1, ga1)
            w3.wait()
            pltpu.async_copy(b_hbm.at[dstv.at[j + 3]], rb1, gb1)

    oz0 = pl.ds((base + cpw - 2) * CHUNK, CHUNK)
    oz1 = pl.ds((base + cpw - 1) * CHUNK, CHUNK)
    pltpu.make_async_copy(ra0, sa_hbm.at[oz0], wa0).wait()
    pltpu.make_async_copy(rb0, sb_hbm.at[oz0], wb0).wait()
    pltpu.make_async_copy(ra1, sa_hbm.at[oz1], wa1).wait()
    pltpu.make_async_copy(rb1, sb_hbm.at[oz1], wb1).wait()


def _sc_edge_gather(A, B, src_m, dst_m):
    f = functools.partial(
        pl.kernel,
        out_type=(jax.ShapeDtypeStruct((EP, H), jnp.float32),
                  jax.ShapeDtypeStruct((EP, H), jnp.float32)),
        mesh=_VMESH,
        scratch_types=[
            pltpu.VMEM((CPWMAX, CHUNK), jnp.int32),
            pltpu.VMEM((CPWMAX, CHUNK), jnp.int32),
            pltpu.VMEM((CHUNK, H), jnp.float32),
            pltpu.VMEM((CHUNK, H), jnp.float32),
            pltpu.VMEM((CHUNK, H), jnp.float32),
            pltpu.VMEM((CHUNK, H), jnp.float32),
        ] + [pltpu.SemaphoreType.DMA] * 8,
    )(_egather_body)
    return f(A, B, src_m, dst_m)


def kernel(x, edge_index, edge_attr, enc_W, enc_b, conv_W, conv_b, ln_g, ln_b,
           no1_W, no1_b, no2_W, no2_b, eo1_W, eo1_b, eo2_W, eo2_b):
    src = edge_index[0]
    dst = edge_index[1]
    pad = jnp.full((EP - E,), SENT, jnp.int32)
    src_f = jnp.concatenate([src, pad])
    src_m = src_f.reshape(NROW, CHUNK)
    dst_m = jnp.concatenate([dst, pad]).reshape(NROW, CHUNK)
    attr_p = jnp.concatenate(
        [edge_attr, jnp.zeros((EP - E, DE), jnp.float32)], axis=0)

    deg_p = _sc_degree(dst_m)
    g, dinv = _tc_prep(deg_p, x, enc_W, enc_b, conv_W[0])
    for i in range(L - 1):
        p = _sc_layer_agg(g, src_f, dst_m)
        g = _tc_combine(p, g, dinv, conv_b[i], ln_g[i], ln_b[i], conv_W[i + 1])
    p = _sc_layer_agg(g, src_f, dst_m)
    A, B, n_out = _tc_final(p, g, dinv, conv_b[L - 1], ln_g[L - 1],
                            ln_b[L - 1], eo1_W[0:H], eo1_W[H:2 * H],
                            no1_W, no1_b, no2_W, no2_b)
    SA, SB = _sc_edge_gather(A, B, src_m, dst_m)
    e_out = _tc_edge(SA, SB, attr_p, eo1_W[2 * H:], eo1_b, eo2_W, eo2_b)
    return (e_out[0:E, 0], n_out)
